# baseline (speedup 1.0000x reference)
import functools

import jax
import jax.numpy as jnp
from jax import lax
from jax.experimental import pallas as pl
from jax.experimental.pallas import tpu as pltpu
from jax.experimental.pallas import tpu_sc as plsc

N_NODES_C = 10000
N_EDGES_C = 320000
D = 128
W = D // 2
LANES = 16


def _zmat_body(x_ref, t_ref, o_ref):
    o_ref[...] = lax.dot_general(
        x_ref[...],
        t_ref[...],
        (((1,), (1,)), ((), ())),
        preferred_element_type=jnp.float32,
        precision=lax.Precision.HIGHEST,
    )


def _z_matmul(x, t0):
    n = x.shape[0]
    blk = 2000
    return pl.pallas_call(
        _zmat_body,
        out_shape=jax.ShapeDtypeStruct((n, D), jnp.float32),
        grid=(n // blk,),
        in_specs=[
            pl.BlockSpec((blk, D), lambda i: (i, 0)),
            pl.BlockSpec((D, D), lambda i: (0, 0)),
        ],
        out_specs=pl.BlockSpec((blk, D), lambda i: (i, 0)),
    )(x, t0)


def _pack_bf16(a):
    return lax.bitcast_convert_type(
        a.astype(jnp.bfloat16).reshape(a.shape[0], W, 2), jnp.int32)


_INFO = plsc.get_sparse_core_info()
_NC = _INFO.num_cores
_NS = _INFO.num_subcores
_NW = _NC * _NS
_EPW = N_EDGES_C // _NW
_CHUNK = 20 * LANES
_NFULL = _EPW // _CHUNK
_REM = _EPW - _NFULL * _CHUNK
_GROUPS = _CHUNK // LANES

_HI_MASK = -65536


def _dot_groups(xr_v, zc_v, ov_v, lane, ngroups):
    def g_body(g, carry):
        e_vec = lane + g * LANES

        def w_body(t, accs, e_vec=e_vec):
            ae0, ae1, ao0, ao1 = accs
            w0 = (lane + (t * 2)) & (W - 1)
            w1 = w0 ^ 1
            xw0 = plsc.load_gather(xr_v, [e_vec, w0])
            zw0 = plsc.load_gather(zc_v, [e_vec, w0])
            xw1 = plsc.load_gather(xr_v, [e_vec, w1])
            zw1 = plsc.load_gather(zc_v, [e_vec, w1])
            xe0 = lax.bitcast_convert_type(xw0 << 16, jnp.float32)
            ze0 = lax.bitcast_convert_type(zw0 << 16, jnp.float32)
            xo0 = lax.bitcast_convert_type(xw0 & _HI_MASK, jnp.float32)
            zo0 = lax.bitcast_convert_type(zw0 & _HI_MASK, jnp.float32)
            xe1 = lax.bitcast_convert_type(xw1 << 16, jnp.float32)
            ze1 = lax.bitcast_convert_type(zw1 << 16, jnp.float32)
            xo1 = lax.bitcast_convert_type(xw1 & _HI_MASK, jnp.float32)
            zo1 = lax.bitcast_convert_type(zw1 & _HI_MASK, jnp.float32)
            return (ae0 + xe0 * ze0, ae1 + xe1 * ze1,
                    ao0 + xo0 * zo0, ao1 + xo1 * zo1)

        zero = jnp.zeros((LANES,), jnp.float32)
        a0, a1, a2, a3 = lax.fori_loop(
            0, W // 2, w_body, (zero, zero, zero, zero), unroll=4)
        m = (a0 + a1) + (a2 + a3)
        t = jnp.exp(jnp.abs(m) * -2.0)
        r = (1.0 - t) / (1.0 + t)
        ov_v[pl.ds(pl.multiple_of(g * LANES, LANES), LANES)] = (
            jnp.where(m < 0.0, -r, r))
        return carry

    lax.fori_loop(0, ngroups, g_body, 0, unroll=False)


def _edge_dot_body(x_hbm, z_hbm, row_hbm, col_hbm, out_hbm,
                   ridx0, ridx1, cidx0, cidx1, xr0, xr1, zc0, zc1, ov0, ov1,
                   x_sp, sem_i0, sem_i1, sem_x0, sem_x1, sem_z0, sem_z1,
                   sem_o0, sem_o1):
    wid = lax.axis_index("s") * _NC + lax.axis_index("c")
    sid = lax.axis_index("s")
    wbase = wid * _EPW
    lane = lax.iota(jnp.int32, LANES)

    ridx = (ridx0, ridx1)
    cidx = (cidx0, cidx1)
    xr = (xr0, xr1)
    zc = (zc0, zc1)
    ov = (ov0, ov1)
    sem_i = (sem_i0, sem_i1)
    sem_x = (sem_x0, sem_x1)
    sem_z = (sem_z0, sem_z1)
    sem_o = (sem_o0, sem_o1)

    rows_per_sub = 624
    sbase = pl.multiple_of(sid * rows_per_sub, 8)
    pltpu.sync_copy(x_hbm.at[pl.ds(sbase, rows_per_sub)],
                    x_sp.at[pl.ds(sbase, rows_per_sub)])

    @pl.when(sid == 0)
    def _():
        pltpu.sync_copy(x_hbm.at[pl.ds(_NS * rows_per_sub, 16)],
                        x_sp.at[pl.ds(_NS * rows_per_sub, 16)])

    plsc.subcore_barrier()

    def issue_idx(c, p):
        off = pl.multiple_of(wbase + c * _CHUNK, 8)
        pltpu.async_copy(row_hbm.at[pl.ds(off, _CHUNK)], ridx[p], sem_i[p])
        pltpu.async_copy(col_hbm.at[pl.ds(off, _CHUNK)], cidx[p], sem_i[p])

    def wait_idx(c, p):
        off = pl.multiple_of(wbase + c * _CHUNK, 8)
        pltpu.make_async_copy(row_hbm.at[pl.ds(off, _CHUNK)], ridx[p],
                              sem_i[p]).wait()
        pltpu.make_async_copy(col_hbm.at[pl.ds(off, _CHUNK)], cidx[p],
                              sem_i[p]).wait()

    _H = _CHUNK // 2

    def issue_rows(p):
        pltpu.async_copy(x_sp.at[ridx[p].at[pl.ds(0, _H)]],
                         xr[p].at[pl.ds(0, _H)], sem_x[p])
        pltpu.async_copy(x_sp.at[ridx[p].at[pl.ds(_H, _H)]],
                         xr[p].at[pl.ds(_H, _H)], sem_x[p])
        pltpu.async_copy(z_hbm.at[cidx[p].at[pl.ds(0, _H)]],
                         zc[p].at[pl.ds(0, _H)], sem_z[p])
        pltpu.async_copy(z_hbm.at[cidx[p].at[pl.ds(_H, _H)]],
                         zc[p].at[pl.ds(_H, _H)], sem_z[p])

    def wait_rows(p):
        pltpu.make_async_copy(x_sp.at[ridx[p].at[pl.ds(0, _H)]],
                              xr[p].at[pl.ds(0, _H)], sem_x[p]).wait()
        pltpu.make_async_copy(x_sp.at[ridx[p].at[pl.ds(_H, _H)]],
                              xr[p].at[pl.ds(_H, _H)], sem_x[p]).wait()
        pltpu.make_async_copy(z_hbm.at[cidx[p].at[pl.ds(0, _H)]],
                              zc[p].at[pl.ds(0, _H)], sem_z[p]).wait()
        pltpu.make_async_copy(z_hbm.at[cidx[p].at[pl.ds(_H, _H)]],
                              zc[p].at[pl.ds(_H, _H)], sem_z[p]).wait()

    def wait_out(c, p):
        off = pl.multiple_of(wbase + c * _CHUNK, 8)
        pltpu.make_async_copy(ov[p], out_hbm.at[pl.ds(off, _CHUNK)],
                              sem_o[p]).wait()

    issue_idx(0, 0)
    issue_idx(1, 1)
    wait_idx(0, 0)
    issue_rows(0)

    def pair_body(j, carry):
        for b in range(2):
            c = j * 2 + b

            @pl.when(c + 1 < _NFULL)
            def _():
                wait_idx(c + 1, 1 - b)
                issue_rows(1 - b)

            wait_rows(b)

            @pl.when(c + 2 < _NFULL)
            def _():
                issue_idx(c + 2, b)

            @pl.when(c >= 2)
            def _():
                wait_out(c - 2, b)

            _dot_groups(xr[b], zc[b], ov[b], lane, _GROUPS)
            off = pl.multiple_of(wbase + c * _CHUNK, 8)
            pltpu.async_copy(ov[b], out_hbm.at[pl.ds(off, _CHUNK)], sem_o[b])
        return carry

    lax.fori_loop(0, _NFULL // 2, pair_body, 0, unroll=False)

    c_last = _NFULL - 1
    wait_rows(0)
    wait_out(c_last - 2, 0)
    wait_out(c_last - 1, 1)
    _dot_groups(xr0, zc0, ov0, lane, _GROUPS)
    off_l = pl.multiple_of(wbase + c_last * _CHUNK, 8)
    pltpu.async_copy(ov0, out_hbm.at[pl.ds(off_l, _CHUNK)], sem_o0)

    tbase = wbase + _EPW - _REM
    ridx_t = ridx1.at[pl.ds(0, _REM)]
    cidx_t = cidx1.at[pl.ds(0, _REM)]
    pltpu.sync_copy(row_hbm.at[pl.ds(tbase, _REM)], ridx_t)
    pltpu.sync_copy(col_hbm.at[pl.ds(tbase, _REM)], cidx_t)
    xr_t = xr1.at[pl.ds(0, _REM)]
    zc_t = zc1.at[pl.ds(0, _REM)]
    cp_x = pltpu.async_copy(x_sp.at[ridx_t], xr_t, sem_x1)
    cp_z = pltpu.async_copy(z_hbm.at[cidx_t], zc_t, sem_z1)
    cp_x.wait()
    cp_z.wait()
    _dot_groups(xr1, zc1, ov1, lane, _REM // LANES)
    wait_out(c_last, 0)
    pltpu.sync_copy(ov1.at[pl.ds(0, _REM)],
                    out_hbm.at[pl.ds(tbase, _REM)])


def _edge_dot(xb, zb, row, col):
    mesh = plsc.VectorSubcoreMesh(core_axis_name="c", subcore_axis_name="s")
    kern = functools.partial(
        pl.kernel,
        mesh=mesh,
        compiler_params=pltpu.CompilerParams(
            needs_layout_passes=False, use_tc_tiling_on_sc=False),
        out_type=jax.ShapeDtypeStruct((N_EDGES_C,), jnp.float32),
        scratch_types=[
            pltpu.VMEM((_CHUNK,), jnp.int32),
            pltpu.VMEM((_CHUNK,), jnp.int32),
            pltpu.VMEM((_CHUNK,), jnp.int32),
            pltpu.VMEM((_CHUNK,), jnp.int32),
            pltpu.VMEM((_CHUNK, W), jnp.int32),
            pltpu.VMEM((_CHUNK, W), jnp.int32),
            pltpu.VMEM((_CHUNK, W), jnp.int32),
            pltpu.VMEM((_CHUNK, W), jnp.int32),
            pltpu.VMEM((_CHUNK,), jnp.float32),
            pltpu.VMEM((_CHUNK,), jnp.float32),
            pltpu.VMEM_SHARED((N_NODES_C, W), jnp.int32),
            pltpu.SemaphoreType.DMA,
            pltpu.SemaphoreType.DMA,
            pltpu.SemaphoreType.DMA,
            pltpu.SemaphoreType.DMA,
            pltpu.SemaphoreType.DMA,
            pltpu.SemaphoreType.DMA,
            pltpu.SemaphoreType.DMA,
            pltpu.SemaphoreType.DMA,
        ],
    )(_edge_dot_body)
    return kern(xb, zb, row, col)


def kernel(x, edge_index, tensor):
    row = edge_index[0].astype(jnp.int32)
    col = edge_index[1].astype(jnp.int32)
    z = _z_matmul(x, tensor[0])
    maps = _edge_dot(_pack_bf16(x), _pack_bf16(z), row, col)
    return maps.reshape(-1, 1)


# speedup vs baseline: 1.0101x; 1.0101x over previous
import functools

import jax
import jax.numpy as jnp
from jax import lax
from jax.experimental import pallas as pl
from jax.experimental.pallas import tpu as pltpu
from jax.experimental.pallas import tpu_sc as plsc

N_NODES_C = 10000
N_EDGES_C = 320000
D = 128
W = D // 2
LANES = 16


def _zmat_body(x_ref, t_ref, o_ref):
    o_ref[...] = lax.dot_general(
        x_ref[...],
        t_ref[...],
        (((1,), (1,)), ((), ())),
        preferred_element_type=jnp.float32,
        precision=lax.Precision.HIGHEST,
    )


def _z_matmul(x, t0):
    n = x.shape[0]
    blk = 2000
    return pl.pallas_call(
        _zmat_body,
        out_shape=jax.ShapeDtypeStruct((n, D), jnp.float32),
        grid=(n // blk,),
        in_specs=[
            pl.BlockSpec((blk, D), lambda i: (i, 0)),
            pl.BlockSpec((D, D), lambda i: (0, 0)),
        ],
        out_specs=pl.BlockSpec((blk, D), lambda i: (i, 0)),
    )(x, t0)


def _pack_bf16(a):
    return lax.bitcast_convert_type(
        a.astype(jnp.bfloat16).reshape(a.shape[0], W, 2), jnp.int32)


_INFO = plsc.get_sparse_core_info()
_NC = _INFO.num_cores
_NS = _INFO.num_subcores
_NW = _NC * _NS
_EPW = N_EDGES_C // _NW
_CHUNK = 20 * LANES
_NFULL = _EPW // _CHUNK
_REM = _EPW - _NFULL * _CHUNK
_GROUPS = _CHUNK // LANES

_HI_MASK = -65536


def _dot_groups(xr_v, zc_v, ov_v, lane, ngroups):
    def g_body(g, carry):
        e_vec = lane + g * LANES

        def w_body(t, accs, e_vec=e_vec):
            ae0, ae1, ao0, ao1 = accs
            w0 = (lane + (t * 2)) & (W - 1)
            w1 = w0 ^ 1
            xw0 = plsc.load_gather(xr_v, [e_vec, w0])
            zw0 = plsc.load_gather(zc_v, [e_vec, w0])
            xw1 = plsc.load_gather(xr_v, [e_vec, w1])
            zw1 = plsc.load_gather(zc_v, [e_vec, w1])
            xe0 = lax.bitcast_convert_type(xw0 << 16, jnp.float32)
            ze0 = lax.bitcast_convert_type(zw0 << 16, jnp.float32)
            xo0 = lax.bitcast_convert_type(xw0 & _HI_MASK, jnp.float32)
            zo0 = lax.bitcast_convert_type(zw0 & _HI_MASK, jnp.float32)
            xe1 = lax.bitcast_convert_type(xw1 << 16, jnp.float32)
            ze1 = lax.bitcast_convert_type(zw1 << 16, jnp.float32)
            xo1 = lax.bitcast_convert_type(xw1 & _HI_MASK, jnp.float32)
            zo1 = lax.bitcast_convert_type(zw1 & _HI_MASK, jnp.float32)
            return (ae0 + xe0 * ze0, ae1 + xe1 * ze1,
                    ao0 + xo0 * zo0, ao1 + xo1 * zo1)

        zero = jnp.zeros((LANES,), jnp.float32)
        a0, a1, a2, a3 = lax.fori_loop(
            0, W // 2, w_body, (zero, zero, zero, zero), unroll=8)
        m = (a0 + a1) + (a2 + a3)
        t = jnp.exp(jnp.abs(m) * -2.0)
        r = (1.0 - t) / (1.0 + t)
        ov_v[pl.ds(pl.multiple_of(g * LANES, LANES), LANES)] = (
            jnp.where(m < 0.0, -r, r))
        return carry

    lax.fori_loop(0, ngroups, g_body, 0, unroll=False)


def _edge_dot_body(x_hbm, z_hbm, row_hbm, col_hbm, out_hbm,
                   ridx0, ridx1, cidx0, cidx1, xr0, xr1, zc0, zc1, ov0, ov1,
                   x_sp, sem_i0, sem_i1, sem_x0, sem_x1, sem_z0, sem_z1,
                   sem_o0, sem_o1):
    wid = lax.axis_index("s") * _NC + lax.axis_index("c")
    sid = lax.axis_index("s")
    wbase = wid * _EPW
    lane = lax.iota(jnp.int32, LANES)

    ridx = (ridx0, ridx1)
    cidx = (cidx0, cidx1)
    xr = (xr0, xr1)
    zc = (zc0, zc1)
    ov = (ov0, ov1)
    sem_i = (sem_i0, sem_i1)
    sem_x = (sem_x0, sem_x1)
    sem_z = (sem_z0, sem_z1)
    sem_o = (sem_o0, sem_o1)

    rows_per_sub = 624
    sbase = pl.multiple_of(sid * rows_per_sub, 8)
    pltpu.sync_copy(x_hbm.at[pl.ds(sbase, rows_per_sub)],
                    x_sp.at[pl.ds(sbase, rows_per_sub)])

    @pl.when(sid == 0)
    def _():
        pltpu.sync_copy(x_hbm.at[pl.ds(_NS * rows_per_sub, 16)],
                        x_sp.at[pl.ds(_NS * rows_per_sub, 16)])

    plsc.subcore_barrier()

    def issue_idx(c, p):
        off = pl.multiple_of(wbase + c * _CHUNK, 8)
        pltpu.async_copy(row_hbm.at[pl.ds(off, _CHUNK)], ridx[p], sem_i[p])
        pltpu.async_copy(col_hbm.at[pl.ds(off, _CHUNK)], cidx[p], sem_i[p])

    def wait_idx(c, p):
        off = pl.multiple_of(wbase + c * _CHUNK, 8)
        pltpu.make_async_copy(row_hbm.at[pl.ds(off, _CHUNK)], ridx[p],
                              sem_i[p]).wait()
        pltpu.make_async_copy(col_hbm.at[pl.ds(off, _CHUNK)], cidx[p],
                              sem_i[p]).wait()

    def issue_rows(p):
        pltpu.async_copy(x_sp.at[ridx[p]], xr[p], sem_x[p])
        pltpu.async_copy(z_hbm.at[cidx[p]], zc[p], sem_z[p])

    def wait_rows(p):
        pltpu.make_async_copy(x_sp.at[ridx[p]], xr[p], sem_x[p]).wait()
        pltpu.make_async_copy(z_hbm.at[cidx[p]], zc[p], sem_z[p]).wait()

    def wait_out(c, p):
        off = pl.multiple_of(wbase + c * _CHUNK, 8)
        pltpu.make_async_copy(ov[p], out_hbm.at[pl.ds(off, _CHUNK)],
                              sem_o[p]).wait()

    issue_idx(0, 0)
    issue_idx(1, 1)
    wait_idx(0, 0)
    issue_rows(0)

    def pair_body(j, carry):
        for b in range(2):
            c = j * 2 + b

            @pl.when(c + 1 < _NFULL)
            def _():
                wait_idx(c + 1, 1 - b)
                issue_rows(1 - b)

            wait_rows(b)

            @pl.when(c + 2 < _NFULL)
            def _():
                issue_idx(c + 2, b)

            @pl.when(c >= 2)
            def _():
                wait_out(c - 2, b)

            _dot_groups(xr[b], zc[b], ov[b], lane, _GROUPS)
            off = pl.multiple_of(wbase + c * _CHUNK, 8)
            pltpu.async_copy(ov[b], out_hbm.at[pl.ds(off, _CHUNK)], sem_o[b])
        return carry

    lax.fori_loop(0, _NFULL // 2, pair_body, 0, unroll=False)

    c_last = _NFULL - 1
    wait_rows(0)
    wait_out(c_last - 2, 0)
    wait_out(c_last - 1, 1)
    _dot_groups(xr0, zc0, ov0, lane, _GROUPS)
    off_l = pl.multiple_of(wbase + c_last * _CHUNK, 8)
    pltpu.async_copy(ov0, out_hbm.at[pl.ds(off_l, _CHUNK)], sem_o0)

    tbase = wbase + _EPW - _REM
    ridx_t = ridx1.at[pl.ds(0, _REM)]
    cidx_t = cidx1.at[pl.ds(0, _REM)]
    pltpu.sync_copy(row_hbm.at[pl.ds(tbase, _REM)], ridx_t)
    pltpu.sync_copy(col_hbm.at[pl.ds(tbase, _REM)], cidx_t)
    xr_t = xr1.at[pl.ds(0, _REM)]
    zc_t = zc1.at[pl.ds(0, _REM)]
    cp_x = pltpu.async_copy(x_sp.at[ridx_t], xr_t, sem_x1)
    cp_z = pltpu.async_copy(z_hbm.at[cidx_t], zc_t, sem_z1)
    cp_x.wait()
    cp_z.wait()
    _dot_groups(xr1, zc1, ov1, lane, _REM // LANES)
    wait_out(c_last, 0)
    pltpu.sync_copy(ov1.at[pl.ds(0, _REM)],
                    out_hbm.at[pl.ds(tbase, _REM)])


def _edge_dot(xb, zb, row, col):
    mesh = plsc.VectorSubcoreMesh(core_axis_name="c", subcore_axis_name="s")
    kern = functools.partial(
        pl.kernel,
        mesh=mesh,
        compiler_params=pltpu.CompilerParams(
            needs_layout_passes=False, use_tc_tiling_on_sc=False),
        out_type=jax.ShapeDtypeStruct((N_EDGES_C,), jnp.float32),
        scratch_types=[
            pltpu.VMEM((_CHUNK,), jnp.int32),
            pltpu.VMEM((_CHUNK,), jnp.int32),
            pltpu.VMEM((_CHUNK,), jnp.int32),
            pltpu.VMEM((_CHUNK,), jnp.int32),
            pltpu.VMEM((_CHUNK, W), jnp.int32),
            pltpu.VMEM((_CHUNK, W), jnp.int32),
            pltpu.VMEM((_CHUNK, W), jnp.int32),
            pltpu.VMEM((_CHUNK, W), jnp.int32),
            pltpu.VMEM((_CHUNK,), jnp.float32),
            pltpu.VMEM((_CHUNK,), jnp.float32),
            pltpu.VMEM_SHARED((N_NODES_C, W), jnp.int32),
            pltpu.SemaphoreType.DMA,
            pltpu.SemaphoreType.DMA,
            pltpu.SemaphoreType.DMA,
            pltpu.SemaphoreType.DMA,
            pltpu.SemaphoreType.DMA,
            pltpu.SemaphoreType.DMA,
            pltpu.SemaphoreType.DMA,
            pltpu.SemaphoreType.DMA,
        ],
    )(_edge_dot_body)
    return kern(xb, zb, row, col)


def kernel(x, edge_index, tensor):
    row = edge_index[0].astype(jnp.int32)
    col = edge_index[1].astype(jnp.int32)
    z = _z_matmul(x, tensor[0])
    maps = _edge_dot(_pack_bf16(x), _pack_bf16(z), row, col)
    return maps.reshape(-1, 1)
